# baseline (speedup 1.0000x reference)
import jax
import jax.numpy as jnp
import numpy as np
from jax.experimental import pallas as pl
from jax.experimental.pallas import tpu as pltpu

B = 128
N_OUT = 96 * 14
KT = 1024
K_SIZES = (16384, 8192, 4096, 2048)
TILES = tuple(k // KT for k in K_SIZES)
STARTS = (0, 16, 24, 28)
ENDS = (16, 24, 28, 30)
NSTEPS = 30
EPS = float(np.finfo(np.float64).eps)


def _body(x0, x1, x2, x3, g, b, w0, w1, w2, w3, out, acc, comb):
    k = pl.program_id(0)
    xs = (x0, x1, x2, x3)
    ws = (w0, w1, w2, w3)
    for e in range(4):
        s, t = STARTS[e], ENDS[e]

        @pl.when(jnp.logical_and(k >= s, k < t))
        def _(e=e, s=s, t=t):
            prod = jnp.dot(xs[e][...], ws[e][...],
                           preferred_element_type=jnp.float32)

            @pl.when(k == s)
            def _():
                acc[...] = prod

            @pl.when(k > s)
            def _():
                acc[...] = acc[...] + prod

            @pl.when(k == t - 1)
            def _(e=e):
                contrib = g[:, e:e + 1] * jnp.exp(acc[...] + b[e:e + 1, :])
                if e == 0:
                    comb[...] = contrib
                else:
                    comb[...] = comb[...] + contrib

    @pl.when(k == NSTEPS - 1)
    def _():
        c = comb[...]
        out[...] = jnp.log(jnp.where(c == 0.0, jnp.float32(EPS), c))


def kernel(xs0, xs1, xs2, xs3, gates, x_dec, W0, b0, W1, b1, W2, b2, W3, b3):
    del x_dec
    xf = [x.reshape(B, -1) for x in (xs0, xs1, xs2, xs3)]
    bstack = jnp.stack([b0, b1, b2, b3], axis=0)

    def x_spec(e):
        nt = TILES[e]
        s = STARTS[e]
        return pl.BlockSpec(
            (B, KT), lambda k, s=s, nt=nt: (0, jnp.clip(k - s, 0, nt - 1)))

    def w_spec(e):
        nt = TILES[e]
        s = STARTS[e]
        return pl.BlockSpec(
            (KT, N_OUT), lambda k, s=s, nt=nt: (jnp.clip(k - s, 0, nt - 1), 0))

    out = pl.pallas_call(
        _body,
        grid=(NSTEPS,),
        in_specs=[
            x_spec(0), x_spec(1), x_spec(2), x_spec(3),
            pl.BlockSpec((B, 4), lambda k: (0, 0)),
            pl.BlockSpec((4, N_OUT), lambda k: (0, 0)),
            w_spec(0), w_spec(1), w_spec(2), w_spec(3),
        ],
        out_specs=pl.BlockSpec((B, N_OUT), lambda k: (0, 0)),
        out_shape=jax.ShapeDtypeStruct((B, N_OUT), jnp.float32),
        scratch_shapes=[
            pltpu.VMEM((B, N_OUT), jnp.float32),
            pltpu.VMEM((B, N_OUT), jnp.float32),
        ],
    )(xf[0], xf[1], xf[2], xf[3], gates, bstack, W0, W1, W2, W3)
    return out.reshape(B, 96, 14)


# speedup vs baseline: 1.0180x; 1.0180x over previous
import jax
import jax.numpy as jnp
import numpy as np
from jax.experimental import pallas as pl
from jax.experimental.pallas import tpu as pltpu

B = 128
N_OUT = 96 * 14
KT = 1024
K_SIZES = (16384, 8192, 4096, 2048)
TILES = tuple(k // KT for k in K_SIZES)
NT = sum(TILES)
RING = 6
EPS = float(np.finfo(np.float64).eps)

_TILE_E = []
_TILE_J = []
for _e, _t in enumerate(TILES):
    for _j in range(_t):
        _TILE_E.append(_e)
        _TILE_J.append(_j)


def _body(x0, x1, x2, x3, g, b, w0, w1, w2, w3, out,
          xbuf, wbuf, acc, comb, xsem, wsem):
    xs = (x0, x1, x2, x3)
    ws = (w0, w1, w2, w3)

    def xcopy(gt):
        e, j, slot = _TILE_E[gt], _TILE_J[gt], gt % RING
        return pltpu.make_async_copy(
            xs[e].at[:, pl.ds(j * KT, KT)], xbuf.at[slot], xsem.at[slot])

    def wcopy(gt):
        e, j, slot = _TILE_E[gt], _TILE_J[gt], gt % RING
        return pltpu.make_async_copy(
            ws[e].at[pl.ds(j * KT, KT), :], wbuf.at[slot], wsem.at[slot])

    for gt in range(RING - 1):
        xcopy(gt).start()
        wcopy(gt).start()

    for gt in range(NT):
        if gt + RING - 1 < NT:
            xcopy(gt + RING - 1).start()
            wcopy(gt + RING - 1).start()
        xcopy(gt).wait()
        wcopy(gt).wait()
        slot = gt % RING
        prod = jnp.dot(xbuf[slot], wbuf[slot],
                       preferred_element_type=jnp.float32)
        e, j = _TILE_E[gt], _TILE_J[gt]
        if j == 0:
            acc[...] = prod
        else:
            acc[...] = acc[...] + prod
        if j == TILES[e] - 1:
            contrib = g[:, e:e + 1] * jnp.exp(acc[...] + b[e:e + 1, :])
            if e == 0:
                comb[...] = contrib
            else:
                comb[...] = comb[...] + contrib

    c = comb[...]
    out[...] = jnp.log(jnp.where(c == 0.0, jnp.float32(EPS), c))


def kernel(xs0, xs1, xs2, xs3, gates, x_dec, W0, b0, W1, b1, W2, b2, W3, b3):
    del x_dec
    xf = [x.reshape(B, -1) for x in (xs0, xs1, xs2, xs3)]
    bstack = jnp.stack([b0, b1, b2, b3], axis=0)

    any_spec = pl.BlockSpec(memory_space=pl.ANY)
    vmem_spec = pl.BlockSpec(memory_space=pltpu.MemorySpace.VMEM)

    out = pl.pallas_call(
        _body,
        in_specs=[any_spec] * 4 + [vmem_spec, vmem_spec] + [any_spec] * 4,
        out_specs=vmem_spec,
        out_shape=jax.ShapeDtypeStruct((B, N_OUT), jnp.float32),
        scratch_shapes=[
            pltpu.VMEM((RING, B, KT), jnp.float32),
            pltpu.VMEM((RING, KT, N_OUT), jnp.float32),
            pltpu.VMEM((B, N_OUT), jnp.float32),
            pltpu.VMEM((B, N_OUT), jnp.float32),
            pltpu.SemaphoreType.DMA((RING,)),
            pltpu.SemaphoreType.DMA((RING,)),
        ],
    )(xf[0], xf[1], xf[2], xf[3], gates, bstack, W0, W1, W2, W3)
    return out.reshape(B, 96, 14)
